# Initial kernel scaffold; baseline (speedup 1.0000x reference)
import jax
import jax.numpy as jnp
from jax.experimental import pallas as pl


def kernel(text, emb_table, w_wide, b_wide, w1, b1, w2, b2, w_out, b_out):
    raise NotImplementedError("write your pallas kernel here")


# speedup vs baseline: 2.6198x; 2.6198x over previous
import functools

import jax
import jax.numpy as jnp
from jax import lax
from jax.experimental import pallas as pl
from jax.experimental.pallas import tpu as pltpu
from jax.experimental.pallas import tpu_sc as plsc

B = 16384
S = 200
D = 64
H1 = 128
H2 = 64

NC = 2
NS = 16
NW = NC * NS
BPW = B // NW
CB = 8
NCHUNK = BPW // CB
CI = CB * S


def _pool_body(text_hbm, table_hbm, out_hbm, idx_v, rows_v, acc_v, sem):
    wid = lax.axis_index("s") * NC + lax.axis_index("c")
    base = wid * BPW

    def chunk(g, carry):
        row0 = base + g * CB
        pltpu.sync_copy(text_hbm.at[pl.ds(row0 * S, CI)], idx_v)
        pltpu.async_copy(table_hbm.at[idx_v], rows_v, sem).wait()
        for b in range(CB):
            def jstep(j2, accs):
                r = b * S + j2 * 4
                new = []
                for k in range(2):
                    for c in range(4):
                        a = accs[k * 4 + c]
                        a = a + rows_v[r + 2 * k, pl.ds(c * 16, 16)]
                        a = a + rows_v[r + 2 * k + 1, pl.ds(c * 16, 16)]
                        new.append(a)
                return tuple(new)

            zero = jnp.zeros((16,), jnp.float32)
            accs = lax.fori_loop(0, S // 4, jstep, (zero,) * 8)
            for c in range(4):
                acc_v[b, pl.ds(c * 16, 16)] = (accs[c] + accs[4 + c]) * (1.0 / S)
        pltpu.sync_copy(acc_v, out_hbm.at[pl.ds(row0, CB)])
        return carry

    lax.fori_loop(0, NCHUNK, chunk, 0)


_pool = pl.kernel(
    _pool_body,
    out_type=jax.ShapeDtypeStruct((B, D), jnp.float32),
    mesh=plsc.VectorSubcoreMesh(core_axis_name="c", subcore_axis_name="s"),
    scratch_types=[
        pltpu.VMEM((CI,), jnp.int32),
        pltpu.VMEM((CI, D), jnp.float32),
        pltpu.VMEM((CB, D), jnp.float32),
        pltpu.SemaphoreType.DMA,
    ],
    compiler_params=pltpu.CompilerParams(use_tc_tiling_on_sc=False),
)


BLK = 2048


def _mlp_body(x_ref, ww_ref, bw_ref, w1_ref, b1_ref, w2_ref, b2_ref,
              wo_ref, bo_ref, o_ref):
    x = x_ref[...]
    wide = jnp.dot(x, ww_ref[...], preferred_element_type=jnp.float32)
    h = jnp.maximum(jnp.dot(x, w1_ref[...], preferred_element_type=jnp.float32)
                    + b1_ref[...], 0.0)
    h = jnp.maximum(jnp.dot(h, w2_ref[...], preferred_element_type=jnp.float32)
                    + b2_ref[...], 0.0)
    o_ref[...] = (jnp.dot(h, wo_ref[...], preferred_element_type=jnp.float32)
                  + bo_ref[...] + wide + bw_ref[...])


def _mlp(pooled, w_wide, b_wide, w1, b1, w2, b2, w_out, b_out):
    full = lambda i: (0, 0)
    return pl.pallas_call(
        _mlp_body,
        grid=(B // BLK,),
        in_specs=[
            pl.BlockSpec((BLK, D), lambda i: (i, 0)),
            pl.BlockSpec((D, 1), full),
            pl.BlockSpec((1, 1), full),
            pl.BlockSpec((D, H1), full),
            pl.BlockSpec((1, H1), full),
            pl.BlockSpec((H1, H2), full),
            pl.BlockSpec((1, H2), full),
            pl.BlockSpec((H2, 1), full),
            pl.BlockSpec((1, 1), full),
        ],
        out_specs=pl.BlockSpec((BLK, 1), lambda i: (i, 0)),
        out_shape=jax.ShapeDtypeStruct((B, 1), jnp.float32),
    )(pooled, w_wide, b_wide.reshape(1, 1), w1, b1.reshape(1, H1),
      w2, b2.reshape(1, H2), w_out, b_out.reshape(1, 1))


def kernel(text, emb_table, w_wide, b_wide, w1, b1, w2, b2, w_out, b_out):
    idx = text.reshape(-1).astype(jnp.int32)
    pooled = _pool(idx, emb_table)
    return _mlp(pooled, w_wide, b_wide, w1, b1, w2, b2, w_out, b_out)


# speedup vs baseline: 3.1985x; 1.2209x over previous
import functools

import jax
import jax.numpy as jnp
from jax import lax
from jax.experimental import pallas as pl
from jax.experimental.pallas import tpu as pltpu
from jax.experimental.pallas import tpu_sc as plsc

B = 16384
S = 200
D = 64
H1 = 128
H2 = 64

NC = 2
NS = 16
NW = NC * NS
BPW = B // NW
CB = 4
NCHUNK = BPW // CB
CI = CB * S


def _accum(rows_v, acc_v, acc_row0):
    for b in range(CB):
        def jstep(j2, accs):
            r = b * S + j2 * 4
            new = []
            for k in range(2):
                for c in range(4):
                    a = accs[k * 4 + c]
                    a = a + rows_v[r + 2 * k, pl.ds(c * 16, 16)]
                    a = a + rows_v[r + 2 * k + 1, pl.ds(c * 16, 16)]
                    new.append(a)
            return tuple(new)

        zero = jnp.zeros((16,), jnp.float32)
        accs = lax.fori_loop(0, S // 4, jstep, (zero,) * 8)
        for c in range(4):
            acc_v[acc_row0 + b, pl.ds(c * 16, 16)] = (
                (accs[c] + accs[4 + c]) * (1.0 / S))


def _pool_body(text_hbm, table_hbm, out_hbm,
               idx0_v, idx1_v, rows0_v, rows1_v, acc_v, sem0, sem1):
    wid = lax.axis_index("s") * NC + lax.axis_index("c")
    base = wid * BPW

    pltpu.sync_copy(text_hbm.at[pl.ds(base * S, CI)], idx0_v)
    cp0 = pltpu.async_copy(table_hbm.at[idx0_v], rows0_v, sem0)

    def step(g2, carry):
        c0 = 2 * g2
        row0 = base + c0 * CB
        pltpu.sync_copy(text_hbm.at[pl.ds((row0 + CB) * S, CI)], idx1_v)
        cp1 = pltpu.async_copy(table_hbm.at[idx1_v], rows1_v, sem1)
        pltpu.make_async_copy(table_hbm.at[idx0_v], rows0_v, sem0).wait()
        _accum(rows0_v, acc_v, 0)
        @pl.when(g2 < NCHUNK // 2 - 1)
        def _():
            pltpu.sync_copy(text_hbm.at[pl.ds((row0 + 2 * CB) * S, CI)],
                            idx0_v)
            pltpu.async_copy(table_hbm.at[idx0_v], rows0_v, sem0)
        cp1.wait()
        _accum(rows1_v, acc_v, CB)
        pltpu.sync_copy(acc_v, out_hbm.at[pl.ds(row0, 2 * CB)])
        return carry

    lax.fori_loop(0, NCHUNK // 2, step, 0)


_pool = pl.kernel(
    _pool_body,
    out_type=jax.ShapeDtypeStruct((B, D), jnp.float32),
    mesh=plsc.VectorSubcoreMesh(core_axis_name="c", subcore_axis_name="s"),
    scratch_types=[
        pltpu.VMEM((CI,), jnp.int32),
        pltpu.VMEM((CI,), jnp.int32),
        pltpu.VMEM((CI, D), jnp.float32),
        pltpu.VMEM((CI, D), jnp.float32),
        pltpu.VMEM((2 * CB, D), jnp.float32),
        pltpu.SemaphoreType.DMA,
        pltpu.SemaphoreType.DMA,
    ],
    compiler_params=pltpu.CompilerParams(use_tc_tiling_on_sc=False),
)


BLK = 2048


def _mlp_body(x_ref, ww_ref, bw_ref, w1_ref, b1_ref, w2_ref, b2_ref,
              wo_ref, bo_ref, o_ref):
    x = x_ref[...]
    wide = jnp.dot(x, ww_ref[...], preferred_element_type=jnp.float32)
    h = jnp.maximum(jnp.dot(x, w1_ref[...], preferred_element_type=jnp.float32)
                    + b1_ref[...], 0.0)
    h = jnp.maximum(jnp.dot(h, w2_ref[...], preferred_element_type=jnp.float32)
                    + b2_ref[...], 0.0)
    o_ref[...] = (jnp.dot(h, wo_ref[...], preferred_element_type=jnp.float32)
                  + bo_ref[...] + wide + bw_ref[...])


def _mlp(pooled, w_wide, b_wide, w1, b1, w2, b2, w_out, b_out):
    full = lambda i: (0, 0)
    return pl.pallas_call(
        _mlp_body,
        grid=(B // BLK,),
        in_specs=[
            pl.BlockSpec((BLK, D), lambda i: (i, 0)),
            pl.BlockSpec((D, 1), full),
            pl.BlockSpec((1, 1), full),
            pl.BlockSpec((D, H1), full),
            pl.BlockSpec((1, H1), full),
            pl.BlockSpec((H1, H2), full),
            pl.BlockSpec((1, H2), full),
            pl.BlockSpec((H2, 1), full),
            pl.BlockSpec((1, 1), full),
        ],
        out_specs=pl.BlockSpec((BLK, 1), lambda i: (i, 0)),
        out_shape=jax.ShapeDtypeStruct((B, 1), jnp.float32),
    )(pooled, w_wide, b_wide.reshape(1, 1), w1, b1.reshape(1, H1),
      w2, b2.reshape(1, H2), w_out, b_out.reshape(1, 1))


def kernel(text, emb_table, w_wide, b_wide, w1, b1, w2, b2, w_out, b_out):
    idx = text.reshape(-1).astype(jnp.int32)
    pooled = _pool(idx, emb_table)
    return _mlp(pooled, w_wide, b_wide, w1, b1, w2, b2, w_out, b_out)


# speedup vs baseline: 4.6486x; 1.4533x over previous
import functools

import jax
import jax.numpy as jnp
from jax import lax
from jax.experimental import pallas as pl
from jax.experimental.pallas import tpu as pltpu
from jax.experimental.pallas import tpu_sc as plsc

B = 16384
VOCAB = 1000000
S = 200
D = 64
H1 = 128
H2 = 64

NC = 2
NS = 16
NW = NC * NS
BPW = B // NW
CB = 4
NCHUNK = BPW // CB
CI = CB * S


def _accum(rows_v, acc_v, acc_row0):
    for b in range(CB):
        def jstep(j2, accs):
            r = b * S + j2 * 4
            new = []
            for k in range(2):
                for c in range(4):
                    a = accs[k * 4 + c]
                    a = a + rows_v[r + 2 * k, pl.ds(c * 16, 16)]
                    a = a + rows_v[r + 2 * k + 1, pl.ds(c * 16, 16)]
                    new.append(a)
            return tuple(new)

        zero = jnp.zeros((16,), jnp.float32)
        accs = lax.fori_loop(0, S // 4, jstep, (zero,) * 8)
        for c in range(4):
            acc_v[acc_row0 + b, pl.ds(c * 16, 16)] = (
                (accs[c] + accs[4 + c]) * (1.0 / S))


def _pool_body(text_hbm, table_hbm, out_hbm,
               idx0_v, idx1_v, rows0_v, rows1_v, acc_v, sem0, sem1):
    wid = lax.axis_index("s") * NC + lax.axis_index("c")
    base = wid * BPW

    pltpu.sync_copy(text_hbm.at[pl.ds(base * S, CI)], idx0_v)
    cp0 = pltpu.async_copy(table_hbm.at[idx0_v], rows0_v, sem0)

    def step(g2, carry):
        c0 = 2 * g2
        row0 = base + c0 * CB
        pltpu.sync_copy(text_hbm.at[pl.ds((row0 + CB) * S, CI)], idx1_v)
        cp1 = pltpu.async_copy(table_hbm.at[idx1_v], rows1_v, sem1)
        pltpu.make_async_copy(table_hbm.at[idx0_v], rows0_v, sem0).wait()
        _accum(rows0_v, acc_v, 0)
        @pl.when(g2 < NCHUNK // 2 - 1)
        def _():
            pltpu.sync_copy(text_hbm.at[pl.ds((row0 + 2 * CB) * S, CI)],
                            idx0_v)
            pltpu.async_copy(table_hbm.at[idx0_v], rows0_v, sem0)
        cp1.wait()
        _accum(rows1_v, acc_v, CB)
        pltpu.sync_copy(acc_v, out_hbm.at[pl.ds(row0, 2 * CB)])
        return carry

    lax.fori_loop(0, NCHUNK // 2, step, 0)


_pool = pl.kernel(
    _pool_body,
    out_type=jax.ShapeDtypeStruct((B, D), jnp.float32),
    mesh=plsc.VectorSubcoreMesh(core_axis_name="c", subcore_axis_name="s"),
    scratch_types=[
        pltpu.VMEM((CI,), jnp.int32),
        pltpu.VMEM((CI,), jnp.int32),
        pltpu.VMEM((CI, D), jnp.float32),
        pltpu.VMEM((CI, D), jnp.float32),
        pltpu.VMEM((2 * CB, D), jnp.float32),
        pltpu.SemaphoreType.DMA,
        pltpu.SemaphoreType.DMA,
    ],
    compiler_params=pltpu.CompilerParams(use_tc_tiling_on_sc=False),
)


TRB = 4096
TRH = TRB // 2
NTRB = (VOCAB + TRB - 1) // TRB
VPAD = NTRB * TRB


def _tr_body(x_ref, o_ref):
    x = x_ref[...]
    o_ref[...] = jnp.concatenate([x[:, :TRH], x[:, TRH:]], axis=0).T


_tr = pl.pallas_call(
    _tr_body,
    grid=(NTRB,),
    in_specs=[pl.BlockSpec((D, TRB), lambda i: (0, i))],
    out_specs=pl.BlockSpec((TRH, 2 * D), lambda i: (i, 0)),
    out_shape=jax.ShapeDtypeStruct((NTRB * TRH, 2 * D), jnp.float32),
)


BLK = 2048


def _mlp_body(x_ref, ww_ref, bw_ref, w1_ref, b1_ref, w2_ref, b2_ref,
              wo_ref, bo_ref, o_ref):
    x = x_ref[...]
    wide = jnp.dot(x, ww_ref[...], preferred_element_type=jnp.float32)
    h = jnp.maximum(jnp.dot(x, w1_ref[...], preferred_element_type=jnp.float32)
                    + b1_ref[...], 0.0)
    h = jnp.maximum(jnp.dot(h, w2_ref[...], preferred_element_type=jnp.float32)
                    + b2_ref[...], 0.0)
    o_ref[...] = (jnp.dot(h, wo_ref[...], preferred_element_type=jnp.float32)
                  + bo_ref[...] + wide + bw_ref[...])


def _mlp(pooled, w_wide, b_wide, w1, b1, w2, b2, w_out, b_out):
    full = lambda i: (0, 0)
    return pl.pallas_call(
        _mlp_body,
        grid=(B // BLK,),
        in_specs=[
            pl.BlockSpec((BLK, D), lambda i: (i, 0)),
            pl.BlockSpec((D, 1), full),
            pl.BlockSpec((1, 1), full),
            pl.BlockSpec((D, H1), full),
            pl.BlockSpec((1, H1), full),
            pl.BlockSpec((H1, H2), full),
            pl.BlockSpec((1, H2), full),
            pl.BlockSpec((H2, 1), full),
            pl.BlockSpec((1, 1), full),
        ],
        out_specs=pl.BlockSpec((BLK, 1), lambda i: (i, 0)),
        out_shape=jax.ShapeDtypeStruct((B, 1), jnp.float32),
    )(pooled, w_wide, b_wide.reshape(1, 1), w1, b1.reshape(1, H1),
      w2, b2.reshape(1, H2), w_out, b_out.reshape(1, 1))


def kernel(text, emb_table, w_wide, b_wide, w1, b1, w2, b2, w_out, b_out):
    idx = text.reshape(-1).astype(jnp.int32)
    idxp = (idx & -TRB) | ((idx & (TRH - 1)) << 1) | ((idx >> 11) & 1)
    t128 = _tr(emb_table.T)
    pooled = _pool(idxp, t128.reshape(VPAD, D))
    return _mlp(pooled, w_wide, b_wide, w1, b1, w2, b2, w_out, b_out)


# speedup vs baseline: 5.3838x; 1.1582x over previous
import functools

import jax
import jax.numpy as jnp
from jax import lax
from jax.experimental import pallas as pl
from jax.experimental.pallas import tpu as pltpu
from jax.experimental.pallas import tpu_sc as plsc

B = 16384
VOCAB = 1000000
S = 200
D = 64
H1 = 128
H2 = 64

NC = 2
NS = 16
NW = NC * NS
BPW = B // NW
CB = 4
NCHUNK = BPW // CB
CI = CB * S


def _accum(rows_v, acc_v, acc_row0):
    for b in range(CB):
        def jstep(j2, accs):
            r = b * S + j2 * 4
            new = []
            for k in range(2):
                for c in range(4):
                    a = accs[k * 4 + c]
                    a = a + rows_v[r + 2 * k, pl.ds(c * 16, 16)]
                    a = a + rows_v[r + 2 * k + 1, pl.ds(c * 16, 16)]
                    new.append(a)
            return tuple(new)

        zero = jnp.zeros((16,), jnp.float32)
        accs = lax.fori_loop(0, S // 4, jstep, (zero,) * 8)
        for c in range(4):
            acc_v[acc_row0 + b, pl.ds(c * 16, 16)] = (
                (accs[c] + accs[4 + c]) * (1.0 / S))


def _pool_body(text_hbm, table_hbm, out_hbm,
               idx0_v, idx1_v, rows0_v, rows1_v, acc_v, sem0, sem1):
    wid = lax.axis_index("s") * NC + lax.axis_index("c")
    base = wid * BPW

    pltpu.sync_copy(text_hbm.at[pl.ds(base * S, CI)], idx0_v)
    cp0 = pltpu.async_copy(table_hbm.at[idx0_v], rows0_v, sem0)

    def step(g2, carry):
        c0 = 2 * g2
        row0 = base + c0 * CB
        pltpu.sync_copy(text_hbm.at[pl.ds((row0 + CB) * S, CI)], idx1_v)
        cp1 = pltpu.async_copy(table_hbm.at[idx1_v], rows1_v, sem1)
        pltpu.make_async_copy(table_hbm.at[idx0_v], rows0_v, sem0).wait()
        _accum(rows0_v, acc_v, 0)
        @pl.when(g2 < NCHUNK // 2 - 1)
        def _():
            pltpu.sync_copy(text_hbm.at[pl.ds((row0 + 2 * CB) * S, CI)],
                            idx0_v)
            pltpu.async_copy(table_hbm.at[idx0_v], rows0_v, sem0)
        cp1.wait()
        _accum(rows1_v, acc_v, CB)
        pltpu.sync_copy(acc_v, out_hbm.at[pl.ds(row0, 2 * CB)])
        return carry

    lax.fori_loop(0, NCHUNK // 2, step, 0)


_pool = pl.kernel(
    _pool_body,
    out_type=jax.ShapeDtypeStruct((B, D), jnp.float32),
    mesh=plsc.VectorSubcoreMesh(core_axis_name="c", subcore_axis_name="s"),
    scratch_types=[
        pltpu.VMEM((CI,), jnp.int32),
        pltpu.VMEM((CI,), jnp.int32),
        pltpu.VMEM((CI, D), jnp.float32),
        pltpu.VMEM((CI, D), jnp.float32),
        pltpu.VMEM((2 * CB, D), jnp.float32),
        pltpu.SemaphoreType.DMA,
        pltpu.SemaphoreType.DMA,
    ],
    compiler_params=pltpu.CompilerParams(use_tc_tiling_on_sc=False),
)


TRB = 16384
TRH = TRB // 2
NTRB = (VOCAB + TRB - 1) // TRB
SH = TRH.bit_length() - 1
VPAD = NTRB * TRB


def _tr_body(x_ref, o_ref):
    x = x_ref[...]
    o_ref[...] = jnp.concatenate([x[:, :TRH], x[:, TRH:]], axis=0).T


_tr = pl.pallas_call(
    _tr_body,
    grid=(NTRB,),
    in_specs=[pl.BlockSpec((D, TRB), lambda i: (0, i))],
    out_specs=pl.BlockSpec((TRH, 2 * D), lambda i: (i, 0)),
    out_shape=jax.ShapeDtypeStruct((NTRB * TRH, 2 * D), jnp.float32),
)


BLK = 2048


def _mlp_body(x_ref, ww_ref, bw_ref, w1_ref, b1_ref, w2_ref, b2_ref,
              wo_ref, bo_ref, o_ref):
    x = x_ref[...]
    wide = jnp.dot(x, ww_ref[...], preferred_element_type=jnp.float32)
    h = jnp.maximum(jnp.dot(x, w1_ref[...], preferred_element_type=jnp.float32)
                    + b1_ref[...], 0.0)
    h = jnp.maximum(jnp.dot(h, w2_ref[...], preferred_element_type=jnp.float32)
                    + b2_ref[...], 0.0)
    o_ref[...] = (jnp.dot(h, wo_ref[...], preferred_element_type=jnp.float32)
                  + bo_ref[...] + wide + bw_ref[...])


def _mlp(pooled, w_wide, b_wide, w1, b1, w2, b2, w_out, b_out):
    full = lambda i: (0, 0)
    return pl.pallas_call(
        _mlp_body,
        grid=(B // BLK,),
        in_specs=[
            pl.BlockSpec((BLK, D), lambda i: (i, 0)),
            pl.BlockSpec((D, 1), full),
            pl.BlockSpec((1, 1), full),
            pl.BlockSpec((D, H1), full),
            pl.BlockSpec((1, H1), full),
            pl.BlockSpec((H1, H2), full),
            pl.BlockSpec((1, H2), full),
            pl.BlockSpec((H2, 1), full),
            pl.BlockSpec((1, 1), full),
        ],
        out_specs=pl.BlockSpec((BLK, 1), lambda i: (i, 0)),
        out_shape=jax.ShapeDtypeStruct((B, 1), jnp.float32),
    )(pooled, w_wide, b_wide.reshape(1, 1), w1, b1.reshape(1, H1),
      w2, b2.reshape(1, H2), w_out, b_out.reshape(1, 1))


def kernel(text, emb_table, w_wide, b_wide, w1, b1, w2, b2, w_out, b_out):
    idx = text.reshape(-1).astype(jnp.int32)
    idxp = (idx & -TRB) | ((idx & (TRH - 1)) << 1) | ((idx >> SH) & 1)
    t128 = _tr(emb_table.T)
    pooled = _pool(idxp, t128.reshape(VPAD, D))
    return _mlp(pooled, w_wide, b_wide, w1, b1, w2, b2, w_out, b_out)
